# baseline (speedup 1.0000x reference)
import functools

import jax
import jax.numpy as jnp
from jax import lax
from jax.experimental import pallas as pl
from jax.experimental.pallas import tpu as pltpu
from jax.experimental.pallas import tpu_sc as plsc

N = 10000
E = 160000
H = 256
C = 8
G = 64

NS = 16
NP = 10240
CH = NP // NS
EPT = 10240
EPT_L = E - 15 * EPT
NV_E = EPT // 16
NV_E_L = EPT_L // 16
NV_C = CH // 16
GP = 128
NV_G = GP // 16

_f32 = jnp.float32


def _rsqrt16(x):
    i = plsc.bitcast(x, jnp.int32)
    i = jnp.full((16,), 0x5F3759DF, jnp.int32) - lax.shift_right_logical(
        i, jnp.full((16,), 1, jnp.int32))
    y = plsc.bitcast(i, _f32)
    for _ in range(3):
        y = y * (1.5 - 0.5 * x * y * y)
    return y


def _sc_graph():
    mesh = plsc.VectorSubcoreMesh(
        core_axis_name="c", subcore_axis_name="s", num_cores=2, num_subcores=NS)

    @functools.partial(
        pl.kernel,
        out_type=(jax.ShapeDtypeStruct((G,), _f32),
                  jax.ShapeDtypeStruct((G,), _f32)),
        mesh=mesh,
        compiler_params=pltpu.CompilerParams(needs_layout_passes=False),
        scratch_types=[
            pltpu.VMEM((2, EPT), jnp.int32),
            pltpu.VMEM((CH,), jnp.int32),
            pltpu.VMEM((NP,), _f32),
            pltpu.VMEM((NP,), _f32),
            pltpu.VMEM((NP,), _f32),
            pltpu.VMEM((NS, CH), _f32),
            pltpu.VMEM((NS, CH), _f32),
            pltpu.VMEM((CH,), _f32),
            pltpu.VMEM((CH,), _f32),
            pltpu.VMEM((CH,), _f32),
            pltpu.VMEM((GP,), _f32),
            pltpu.VMEM((GP,), _f32),
            pltpu.VMEM((NS, GP), _f32),
            pltpu.SemaphoreType.DMA,
            pltpu.SemaphoreType.DMA,
            pltpu.SemaphoreType.DMA,
            pltpu.VMEM_SHARED((NS, NP), _f32),
            pltpu.VMEM_SHARED((NS, NP), _f32),
            pltpu.VMEM_SHARED((NP,), _f32),
            pltpu.VMEM_SHARED((NS, GP), _f32),
            pltpu.VMEM_SHARED((NS, GP), _f32),
        ],
    )
    def run(ei_h, gid_h, u_out, c_out,
            ei_v, gid_v, acc_a, acc_b, node_v, slab, slab2, ns_c, nd_c, u_c,
            accG, cntG, slabG, sem_e, sem_g, sem_n,
            mat_a, mat_b, vec_sh, matG, matC):
        s = lax.axis_index("s")
        c = lax.axis_index("c")
        ones16 = jnp.ones((16,), _f32)
        zeros16 = jnp.zeros((16,), _f32)

        def off16(i):
            return pl.ds(pl.multiple_of(i * 16, 16), 16)

        def zero_ref(ref, nv):
            @plsc.parallel_loop(0, nv, 1, unroll=8)
            def _(i):
                ref[off16(i)] = zeros16

        def reduce_rows(mat, nv, out_fn):
            pltpu.sync_copy(mat, slab)

            @plsc.parallel_loop(0, nv, 1, unroll=2)
            def _(i):
                acc = slab[0, off16(i)]
                for r in range(1, NS):
                    acc = acc + slab[r, off16(i)]
                out_fn(i, acc)

        my_nodes = pl.ds(pl.multiple_of(s * CH, 8), CH)

        @pl.when(s < NS - 1)
        def _():
            pltpu.async_copy(
                ei_h.at[:, pl.ds(pl.multiple_of(s * EPT, 512), EPT)], ei_v,
                sem_e)
            pltpu.async_copy(gid_h.at[my_nodes], gid_v, sem_g)

        @pl.when(s == NS - 1)
        def _():
            tail = N - (NS - 1) * CH
            pltpu.async_copy(ei_h.at[:, pl.ds((NS - 1) * EPT, EPT_L)],
                             ei_v.at[:, pl.ds(0, EPT_L)], sem_e)
            pltpu.async_copy(gid_h.at[pl.ds((NS - 1) * CH, tail)],
                             gid_v.at[pl.ds(0, tail)], sem_g)
            for i in range(tail // 16, NV_C):
                gid_v[off16(i)] = jnp.full((16,), G, jnp.int32)

        zero_ref(acc_a, NP // 16)
        zero_ref(acc_b, NP // 16)

        @pl.when(s < NS - 1)
        def _():
            pltpu.make_async_copy(
                ei_h.at[:, pl.ds(pl.multiple_of(s * EPT, 512), EPT)], ei_v,
                sem_e).wait()
            pltpu.make_async_copy(gid_h.at[my_nodes], gid_v, sem_g).wait()

        @pl.when(s == NS - 1)
        def _():
            tail = N - (NS - 1) * CH
            pltpu.make_async_copy(ei_h.at[:, pl.ds((NS - 1) * EPT, EPT_L)],
                                  ei_v.at[:, pl.ds(0, EPT_L)], sem_e).wait()
            pltpu.make_async_copy(gid_h.at[pl.ds((NS - 1) * CH, tail)],
                                  gid_v.at[pl.ds(0, tail)], sem_g).wait()

        def deg_body(i):
            plsc.addupdate_scatter(acc_a, [ei_v[0, off16(i)]], ones16)
            plsc.addupdate_scatter(acc_b, [ei_v[1, off16(i)]], ones16)

        plsc.parallel_loop(0, NV_E_L, 1, unroll=16)(deg_body)

        @pl.when(s < NS - 1)
        def _():
            plsc.parallel_loop(NV_E_L, NV_E, 1, unroll=16)(deg_body)

        pltpu.sync_copy(acc_a, mat_a.at[s])
        pltpu.sync_copy(acc_b, mat_b.at[s])
        plsc.subcore_barrier()

        h_a = pltpu.async_copy(mat_a.at[:, my_nodes], slab, sem_n)
        h_b = pltpu.async_copy(mat_b.at[:, my_nodes], slab2, sem_g)
        h_a.wait()
        h_b.wait()

        @plsc.parallel_loop(0, NV_C, 1, unroll=2)
        def _(i):
            va = slab[0, off16(i)]
            vb = slab2[0, off16(i)]
            for r in range(1, NS):
                va = va + slab[r, off16(i)]
                vb = vb + slab2[r, off16(i)]
            ns = _rsqrt16(jnp.maximum(va, 1.0))
            nd = _rsqrt16(jnp.maximum(vb, 1.0))
            ns_c[off16(i)] = ns
            nd_c[off16(i)] = nd
            u_c[off16(i)] = vb * ns

        pltpu.sync_copy(u_c, vec_sh.at[my_nodes])
        plsc.subcore_barrier()

        h_n = pltpu.async_copy(vec_sh, node_v, sem_n)
        zero_ref(acc_a, NP // 16)
        h_n.wait()

        def edge_pass():
            def body(i):
                v = plsc.load_gather(node_v, [ei_v[0, off16(i)]])
                plsc.addupdate_scatter(acc_a, [ei_v[1, off16(i)]], v)

            plsc.parallel_loop(0, NV_E_L, 1, unroll=16)(body)

            @pl.when(s < NS - 1)
            def _():
                plsc.parallel_loop(NV_E_L, NV_E, 1, unroll=16)(body)

        edge_pass()
        pltpu.sync_copy(acc_a, mat_a.at[s])
        plsc.subcore_barrier()

        def p_fn(i, v):
            u_c[off16(i)] = v * nd_c[off16(i)] * ns_c[off16(i)]
        reduce_rows(mat_a.at[:, my_nodes], NV_C, p_fn)

        pltpu.sync_copy(u_c, vec_sh.at[my_nodes])
        plsc.subcore_barrier()

        h_n2 = pltpu.async_copy(vec_sh, node_v, sem_n)
        zero_ref(acc_a, NP // 16)
        h_n2.wait()
        edge_pass()

        pltpu.sync_copy(acc_a, mat_a.at[s])
        plsc.subcore_barrier()

        def u_fn(i, v):
            u_c[off16(i)] = v * nd_c[off16(i)]
        reduce_rows(mat_a.at[:, my_nodes], NV_C, u_fn)

        zero_ref(accG, NV_G)
        zero_ref(cntG, NV_G)

        @plsc.parallel_loop(0, NV_C, 1, unroll=4)
        def _(i):
            g = gid_v[off16(i)]
            plsc.addupdate_scatter(accG, [g], u_c[off16(i)])
            plsc.addupdate_scatter(cntG, [g], ones16)

        pltpu.sync_copy(accG, matG.at[s])
        pltpu.sync_copy(cntG, matC.at[s])
        plsc.subcore_barrier()

        @pl.when(jnp.logical_and(s == 0, c == 0))
        def _():
            def redG(mat, out_ref):
                pltpu.sync_copy(mat, slabG)

                @plsc.parallel_loop(0, NV_G, 1, unroll=2)
                def _(i):
                    acc = slabG[0, off16(i)]
                    for r in range(1, NS):
                        acc = acc + slabG[r, off16(i)]
                    out_ref[off16(i)] = acc
            redG(matG, accG)
            redG(matC, cntG)
            pltpu.sync_copy(accG.at[pl.ds(0, G)], u_out)
            pltpu.sync_copy(cntG.at[pl.ds(0, G)], c_out)

    return run


def _tc_bvec(W1, W2, Wfc):
    def body(w1_ref, w2_ref, wfc_ref, o_ref):
        r1 = jnp.maximum(w1_ref[...], 0.0)
        q = jnp.maximum(
            jnp.dot(r1, w2_ref[...], preferred_element_type=_f32), 0.0)
        b = jnp.dot(q, wfc_ref[...], preferred_element_type=_f32)
        o_ref[...] = jnp.concatenate([b, jnp.zeros((1, 16 - C), _f32)], axis=1)

    return pl.pallas_call(
        body, out_shape=jax.ShapeDtypeStruct((1, 16), _f32),
    )(W1, W2, Wfc).reshape(16)


def kernel(edge_index, node_graph_ids, W1, W2, Wfc):
    b = _tc_bvec(W1, W2, Wfc)
    u_sum, cnt = _sc_graph()(edge_index, node_graph_ids.astype(jnp.int32))
    a = u_sum / jnp.maximum(cnt, 1.0)
    return a[:, None] * b[None, :C]
